# baseline (speedup 1.0000x reference)
import functools

import jax
import jax.numpy as jnp
from jax import lax
from jax.experimental import pallas as pl
from jax.experimental.pallas import tpu as pltpu
from jax.experimental.pallas import tpu_sc as plsc

_f32 = jnp.float32
_i32 = jnp.int32

NC = 2
NS = 16
NW = NC * NS
L = 16

H = 4
C = 32
G = 512

_DNUMS = lax.GatherDimensionNumbers(
    offset_dims=(), collapsed_slice_dims=(0,), start_index_map=(0,))


def _lane_shuffle(v, idx):
    return lax.gather(v, idx[:, None], _DNUMS, slice_sizes=(1,),
                      mode=lax.GatherScatterMode.PROMISE_IN_BOUNDS)


def _shift4(v):
    io = lax.iota(_i32, L)
    return _lane_shuffle(v, jnp.minimum(io + 4, 8))


def _splat(v, lane):
    return _lane_shuffle(v, lax.iota(_i32, L) * 0 + lane)


def _mesh():
    return plsc.VectorSubcoreMesh(core_axis_name="c", subcore_axis_name="s")


_SC_PARAMS = pltpu.CompilerParams(use_tc_tiling_on_sc=False)
_SC_PARAMS_NL = pltpu.CompilerParams(use_tc_tiling_on_sc=False,
                                     needs_layout_passes=False)


def _zero_vmem(ref, rows, width):
    z = jnp.zeros((L,), _f32)

    @plsc.parallel_loop(0, rows, unroll=4)
    def _(r):
        for kk in range(width // L):
            ref[r, pl.ds(kk * L, L)] = z


@functools.partial(jax.jit, static_argnames=("np_pad", "k"))
def _pass0(idx, tlo, thi, ta, *, np_pad, k):
    nchunks = np_pad // k

    @functools.partial(
        pl.kernel,
        out_type=(jax.ShapeDtypeStruct((np_pad, 64), _f32),
                  jax.ShapeDtypeStruct((np_pad, 64), _f32),
                  jax.ShapeDtypeStruct((np_pad, 16), _f32)),
        mesh=_mesh(),
        compiler_params=_SC_PARAMS,
        scratch_types=[
            pltpu.VMEM((2, k), _i32),
            pltpu.VMEM((2, k, 64), _f32),
            pltpu.VMEM((2, k, 64), _f32),
            pltpu.VMEM((2, k, 16), _f32),
            pltpu.SemaphoreType.DMA,
            pltpu.SemaphoreType.DMA,
        ],
    )
    def kfn(idx_hbm, tlo_hbm, thi_hbm, ta_hbm, lo_out, hi_out, a_out,
            idx_v, lo_v, hi_v, a_v, sg0, sg1):
        w = lax.axis_index("s") * NC + lax.axis_index("c")
        trips = (nchunks - w + NW - 1) // NW
        sgs = (sg0, sg1)

        def prep(ci, b):
            base = (w + ci * NW) * k
            pltpu.sync_copy(idx_hbm.at[pl.ds(base, k)], idx_v.at[b])
            pltpu.async_copy(tlo_hbm.at[idx_v.at[b]], lo_v.at[b], sgs[b])
            pltpu.async_copy(thi_hbm.at[idx_v.at[b]], hi_v.at[b], sgs[b])
            pltpu.async_copy(ta_hbm.at[idx_v.at[b]], a_v.at[b], sgs[b])

        def wait_gathers(b):
            pltpu.make_async_copy(
                tlo_hbm.at[idx_v.at[b]], lo_v.at[b], sgs[b]).wait()
            pltpu.make_async_copy(
                thi_hbm.at[idx_v.at[b]], hi_v.at[b], sgs[b]).wait()
            pltpu.make_async_copy(
                ta_hbm.at[idx_v.at[b]], a_v.at[b], sgs[b]).wait()

        @pl.when(trips > 0)
        def _():
            prep(0, 0)

        def body(p, _):
            for b in range(2):
                ci = 2 * p + b

                @pl.when(ci < trips)
                def _():
                    @pl.when(ci + 1 < trips)
                    def _():
                        prep(ci + 1, 1 - b)

                    wait_gathers(b)
                    base = (w + ci * NW) * k
                    pltpu.sync_copy(lo_v.at[b], lo_out.at[pl.ds(base, k)])
                    pltpu.sync_copy(hi_v.at[b], hi_out.at[pl.ds(base, k)])
                    pltpu.sync_copy(a_v.at[b], a_out.at[pl.ds(base, k)])
            return 0

        lax.fori_loop(0, (trips + 1) // 2, body, 0)

    return kfn(idx, tlo, thi, ta)


@functools.partial(jax.jit, static_argnames=("n", "n_pad", "k"))
def _pass1(src, dst, a_tab, mvec, *, n, n_pad, k):
    e_tot = src.shape[0]
    nchunks = e_tot // k
    rpt = n // NS
    zr = 625
    assert rpt % zr == 0

    @functools.partial(
        pl.kernel,
        out_type=jax.ShapeDtypeStruct((NC, n_pad, 16), _f32),
        mesh=_mesh(),
        compiler_params=_SC_PARAMS,
        scratch_types=[
            pltpu.VMEM((2, k), _i32),
            pltpu.VMEM((2, k), _i32),
            pltpu.VMEM((2, k), _i32),
            pltpu.VMEM((2, k, 16), _f32),
            pltpu.VMEM((2, k, 16), _f32),
            pltpu.VMEM((2, k, 16), _f32),
            pltpu.VMEM((zr, 16), _f32),
            pltpu.VMEM((L,), _f32),
            pltpu.VMEM_SHARED((n, 16), _f32),
            pltpu.SemaphoreType.DMA,
            pltpu.SemaphoreType.DMA,
            pltpu.SemaphoreType.DMA,
            pltpu.SemaphoreType.DMA,
        ],
    )
    def kfn(src_hbm, dst_hbm, a_hbm, m_hbm, den_out,
            src_v, dst_v, dsc_v, va_v, vb_v, t_v, z_v, m_v, den_sh,
            sg0, sg1, ss0, ss1):
        c = lax.axis_index("c")
        s = lax.axis_index("s")
        w = s * NC + c
        sgs = (sg0, sg1)
        sss = (ss0, ss1)
        _zero_vmem(z_v, zr, 16)
        for j in range(rpt // zr):
            pltpu.sync_copy(z_v, den_sh.at[pl.ds(s * rpt + j * zr, zr)])
        pltpu.sync_copy(m_hbm, m_v)
        mv = m_v[...]
        plsc.subcore_barrier()
        trips = (nchunks - w + NW - 1) // NW

        def load_idx(ci, b):
            base = (w + ci * NW) * k
            pltpu.sync_copy(src_hbm.at[pl.ds(base, k)], src_v.at[b])
            pltpu.sync_copy(dst_hbm.at[pl.ds(base, k)], dst_v.at[b])

        def issue_gathers(b):
            pltpu.async_copy(a_hbm.at[src_v.at[b]], va_v.at[b], sgs[b])
            pltpu.async_copy(a_hbm.at[dst_v.at[b]], vb_v.at[b], sgs[b])

        def wait_gathers(b):
            pltpu.make_async_copy(
                a_hbm.at[src_v.at[b]], va_v.at[b], sgs[b]).wait()
            pltpu.make_async_copy(
                a_hbm.at[dst_v.at[b]], vb_v.at[b], sgs[b]).wait()

        def issue_scatter(b):
            pltpu.async_copy(t_v.at[b], den_sh.at[dsc_v.at[b]], sss[b],
                             add=True)

        def wait_scatter(b):
            pltpu.make_async_copy(
                t_v.at[b], den_sh.at[dsc_v.at[b]], sss[b]).wait()

        def compute(b):
            @plsc.parallel_loop(0, k // L, unroll=2)
            def _(r):
                dsc_v[b, pl.ds(r * L, L)] = dst_v[b, pl.ds(r * L, L)]

            @plsc.parallel_loop(0, k, unroll=8)
            def _(e):
                va = va_v[b, e, :]
                vb = vb_v[b, e, :]
                ev = va + _shift4(vb)
                lr = jnp.maximum(ev, 0.2 * ev)
                t_v[b, e, :] = jnp.exp(lr - mv)

        @pl.when(trips > 0)
        def _():
            load_idx(0, 0)
            issue_gathers(0)

        def body(p, _):
            for b in range(2):
                ci = 2 * p + b

                @pl.when(ci < trips)
                def _():
                    @pl.when(ci + 1 < trips)
                    def _():
                        load_idx(ci + 1, 1 - b)
                        issue_gathers(1 - b)

                    wait_gathers(b)

                    @pl.when(ci >= 2)
                    def _():
                        wait_scatter(b)

                    compute(b)
                    issue_scatter(b)
            return 0

        lax.fori_loop(0, (trips + 1) // 2, body, 0)

        def wait_scatter_dyn(j):
            @pl.when(j >= 0)
            def _():
                @pl.when(lax.rem(j, 2) == 0)
                def _():
                    wait_scatter(0)

                @pl.when(lax.rem(j, 2) == 1)
                def _():
                    wait_scatter(1)

        wait_scatter_dyn(trips - 1)
        wait_scatter_dyn(trips - 2)
        plsc.subcore_barrier()
        for j in range(rpt // zr):
            off = s * rpt + j * zr
            pltpu.sync_copy(den_sh.at[pl.ds(off, zr)],
                            den_out.at[c, pl.ds(off, zr)])

    return kfn(src, dst, a_tab, mvec)


@functools.partial(jax.jit, static_argnames=("k",))
def _combine(den, a_tab, *, k):
    n_pad = a_tab.shape[0]
    nchunks = n_pad // k

    @functools.partial(
        pl.kernel,
        out_type=jax.ShapeDtypeStruct((n_pad, 16), _f32),
        mesh=_mesh(),
        compiler_params=_SC_PARAMS,
        scratch_types=[
            pltpu.VMEM((k, 16), _f32),
            pltpu.VMEM((k, 16), _f32),
            pltpu.VMEM((k, 16), _f32),
            pltpu.VMEM((k, 16), _f32),
        ],
    )
    def kfn(den_hbm, a_hbm, d_out, d0_v, d1_v, a_v, o_v):
        w = lax.axis_index("s") * NC + lax.axis_index("c")
        trips = (nchunks - w + NW - 1) // NW
        io = lax.iota(_i32, L)
        iof = io.astype(_f32)
        m47 = jnp.maximum(
            0.0, jnp.minimum(1.0, jnp.minimum(iof - 3.0, 8.0 - iof)))
        sh_r = jnp.maximum(io - 4, 0)

        def body(i, _):
            base = (w + i * NW) * k
            pltpu.sync_copy(den_hbm.at[0, pl.ds(base, k)], d0_v)
            pltpu.sync_copy(den_hbm.at[1, pl.ds(base, k)], d1_v)
            pltpu.sync_copy(a_hbm.at[pl.ds(base, k)], a_v)

            @plsc.parallel_loop(0, k, unroll=4)
            def _(r):
                sm = d0_v[r, :] + d1_v[r, :] + 1e-16
                rr = 1.0 / sm
                rsh = _lane_shuffle(rr, sh_r)
                o_v[r, :] = _shift4(a_v[r, :]) + rsh * m47

            pltpu.sync_copy(o_v, d_out.at[pl.ds(base, k)])
            return 0

        lax.fori_loop(0, trips, body, 0)

    return kfn(den, a_tab)


@functools.partial(jax.jit, static_argnames=("n", "n_pad", "k"))
def _pass2h(src, dst, a_tab, d_tab, hwh, mvec, *, n, n_pad, k):
    e_tot = src.shape[0]
    nchunks = e_tot // k
    rpt = n // NS
    zr = 625
    assert rpt % zr == 0

    @functools.partial(
        pl.kernel,
        out_type=jax.ShapeDtypeStruct((NC, n_pad, 16), _f32),
        mesh=_mesh(),
        compiler_params=_SC_PARAMS,
        scratch_types=[
            pltpu.VMEM((3, k), _i32),
            pltpu.VMEM((3, k), _i32),
            pltpu.VMEM((3, k), _i32),
            pltpu.VMEM((3, k, 16), _f32),
            pltpu.VMEM((3, k, 16), _f32),
            pltpu.VMEM((3, k, 64), _f32),
            pltpu.VMEM((3, k, 16), _f32),
            pltpu.VMEM((zr, 16), _f32),
            pltpu.VMEM((L,), _f32),
            pltpu.VMEM_SHARED((n, 16), _f32),
            pltpu.SemaphoreType.DMA,
            pltpu.SemaphoreType.DMA,
            pltpu.SemaphoreType.DMA,
            pltpu.SemaphoreType.DMA,
            pltpu.SemaphoreType.DMA,
            pltpu.SemaphoreType.DMA,
        ],
    )
    def kfn(src_hbm, dst_hbm, a_hbm, d_hbm, hw_hbm, m_hbm, out_hbm,
            src_v, dst_v, dsc_v, va_v, vd_v, hw_v, c_v, z_v, m_v, acc_sh,
            sg0, sg1, sg2, ss0, ss1, ss2):
        c = lax.axis_index("c")
        s = lax.axis_index("s")
        w = s * NC + c
        sgs = (sg0, sg1, sg2)
        sss = (ss0, ss1, ss2)
        _zero_vmem(z_v, zr, 16)
        for j in range(rpt // zr):
            pltpu.sync_copy(z_v, acc_sh.at[pl.ds(s * rpt + j * zr, zr)])
        pltpu.sync_copy(m_hbm, m_v)
        mv = m_v[...]
        plsc.subcore_barrier()
        trips = (nchunks - w + NW - 1) // NW

        def load_idx(ci, b):
            base = (w + ci * NW) * k
            pltpu.sync_copy(src_hbm.at[pl.ds(base, k)], src_v.at[b])
            pltpu.sync_copy(dst_hbm.at[pl.ds(base, k)], dst_v.at[b])

        def issue_gathers(b):
            pltpu.async_copy(a_hbm.at[src_v.at[b]], va_v.at[b], sgs[b])
            pltpu.async_copy(d_hbm.at[dst_v.at[b]], vd_v.at[b], sgs[b])
            pltpu.async_copy(hw_hbm.at[src_v.at[b]], hw_v.at[b], sgs[b])

        def wait_gathers(b):
            pltpu.make_async_copy(
                a_hbm.at[src_v.at[b]], va_v.at[b], sgs[b]).wait()
            pltpu.make_async_copy(
                d_hbm.at[dst_v.at[b]], vd_v.at[b], sgs[b]).wait()
            pltpu.make_async_copy(
                hw_hbm.at[src_v.at[b]], hw_v.at[b], sgs[b]).wait()

        def issue_scatter(b):
            pltpu.async_copy(c_v.at[b], acc_sh.at[dsc_v.at[b]], sss[b],
                             add=True)

        def wait_scatter(b):
            pltpu.make_async_copy(
                c_v.at[b], acc_sh.at[dsc_v.at[b]], sss[b]).wait()

        def compute(b):
            @plsc.parallel_loop(0, k // L, unroll=2)
            def _(r):
                dsc_v[b, pl.ds(r * L, L)] = dst_v[b, pl.ds(r * L, L)]

            @plsc.parallel_loop(0, k, unroll=8)
            def _(e):
                va = va_v[b, e, :]
                vd = vd_v[b, e, :]
                ev = va + vd
                lr = jnp.maximum(ev, 0.2 * ev)
                t = jnp.exp(lr - mv)
                alpha = t * _shift4(vd)
                acc = _splat(alpha, 0) * hw_v[b, e, pl.ds(0, L)]
                for h in range(1, H):
                    acc = acc + _splat(alpha, h) * hw_v[b, e, pl.ds(h * L, L)]
                c_v[b, e, :] = acc

        @pl.when(trips > 0)
        def _():
            load_idx(0, 0)
            issue_gathers(0)

        def body(p, _):
            for b in range(3):
                ci = 3 * p + b

                @pl.when(ci < trips)
                def _():
                    nb = (b + 1) % 3

                    @pl.when(ci + 1 < trips)
                    def _():
                        load_idx(ci + 1, nb)
                        issue_gathers(nb)

                    wait_gathers(b)

                    @pl.when(ci >= 3)
                    def _():
                        wait_scatter(b)

                    compute(b)
                    issue_scatter(b)
            return 0

        lax.fori_loop(0, (trips + 2) // 3, body, 0)

        def wait_scatter_dyn(j):
            @pl.when(j >= 0)
            def _():
                for b in range(3):
                    @pl.when(lax.rem(j, 3) == b)
                    def _():
                        wait_scatter(b)

        wait_scatter_dyn(trips - 1)
        wait_scatter_dyn(trips - 2)
        wait_scatter_dyn(trips - 3)
        plsc.subcore_barrier()
        for j in range(rpt // zr):
            off = s * rpt + j * zr
            pltpu.sync_copy(acc_sh.at[pl.ds(off, zr)],
                            out_hbm.at[c, pl.ds(off, zr)])

    return kfn(src, dst, a_tab, d_tab, hwh, mvec)


@functools.partial(jax.jit, static_argnames=("n", "k"))
def _pool(s2lo, s2hi, batch, b1, *, n, k):
    n_pad = batch.shape[0]
    nchunks = n_pad // k
    rpt = G // NS

    @functools.partial(
        pl.kernel,
        out_type=jax.ShapeDtypeStruct((NC, G, 32), _f32),
        mesh=_mesh(),
        compiler_params=_SC_PARAMS,
        scratch_types=[
            pltpu.VMEM((k,), _i32),
            pltpu.VMEM((k, 16), _f32),
            pltpu.VMEM((k, 16), _f32),
            pltpu.VMEM((k, 16), _f32),
            pltpu.VMEM((k, 16), _f32),
            pltpu.VMEM((k, 32), _f32),
            pltpu.VMEM((rpt, 32), _f32),
            pltpu.VMEM((32,), _f32),
            pltpu.VMEM_SHARED((G, 32), _f32),
        ],
    )
    def kfn(lo_hbm, hi_hbm, bat_hbm, b1_hbm, out_hbm,
            bat_v, l0_v, l1_v, h0_v, h1_v, h_v, z_v, b_v, acc_sh):
        c = lax.axis_index("c")
        s = lax.axis_index("s")
        w = s * NC + c
        _zero_vmem(z_v, rpt, 32)
        pltpu.sync_copy(z_v, acc_sh.at[pl.ds(s * rpt, rpt)])
        pltpu.sync_copy(b1_hbm, b_v)
        blo = b_v[pl.ds(0, L)]
        bhi = b_v[pl.ds(L, L)]
        plsc.subcore_barrier()
        trips = (nchunks - w + NW - 1) // NW
        inv_h = _f32(1.0 / H)

        def body(i, _):
            base = (w + i * NW) * k
            pltpu.sync_copy(bat_hbm.at[pl.ds(base, k)], bat_v)
            pltpu.sync_copy(lo_hbm.at[0, pl.ds(base, k)], l0_v)
            pltpu.sync_copy(lo_hbm.at[1, pl.ds(base, k)], l1_v)
            pltpu.sync_copy(hi_hbm.at[0, pl.ds(base, k)], h0_v)
            pltpu.sync_copy(hi_hbm.at[1, pl.ds(base, k)], h1_v)

            @plsc.parallel_loop(0, k, unroll=4)
            def _(r):
                valid = jnp.where(base + r < n, _f32(1.0), _f32(0.0))
                lo = (l0_v[r, :] + l1_v[r, :]) * inv_h
                hi = (h0_v[r, :] + h1_v[r, :]) * inv_h
                lo = jnp.maximum(lo + blo, 0.0)
                hi = jnp.maximum(hi + bhi, 0.0)
                h_v[r, pl.ds(0, L)] = lo * valid
                h_v[r, pl.ds(L, L)] = hi * valid
            pltpu.sync_copy(h_v, acc_sh.at[bat_v], add=True)
            return 0

        lax.fori_loop(0, trips, body, 0)
        plsc.subcore_barrier()
        pltpu.sync_copy(acc_sh.at[pl.ds(s * rpt, rpt)],
                        out_hbm.at[c, pl.ds(s * rpt, rpt)])

    return kfn(s2lo, s2hi, batch, b1)


def _tc_layer(slo, shi, b0r, w1lo, w1hi, amlo, amhi, *, n, n_pad, bn=1000):
    def body(sl0_ref, sl1_ref, sh0_ref, sh1_ref, b_ref, wl_ref, wh_ref,
             al_ref, ah_ref, lo_ref, hi_ref, a_ref, mx_ref):
        hmean = jnp.concatenate(
            [sl0_ref[0] + sl1_ref[0], sh0_ref[0] + sh1_ref[0]],
            axis=1) * _f32(1.0 / H)
        h = jnp.maximum(hmean + b_ref[...], 0.0)
        hwlo = jnp.dot(h, wl_ref[...], preferred_element_type=_f32)
        hwhi = jnp.dot(h, wh_ref[...], preferred_element_type=_f32)
        a1 = (jnp.dot(hwlo, al_ref[...], preferred_element_type=_f32)
              + jnp.dot(hwhi, ah_ref[...], preferred_element_type=_f32))
        lo_ref[...] = hwlo
        hi_ref[...] = hwhi
        a_ref[...] = a1
        bm = jnp.broadcast_to(jnp.max(a1, axis=0, keepdims=True), (8, 16))
        i = pl.program_id(0)

        @pl.when(i == 0)
        def _():
            mx_ref[...] = bm

        @pl.when(i > 0)
        def _():
            mx_ref[...] = jnp.maximum(mx_ref[...], bm)

    return pl.pallas_call(
        body,
        grid=(n // bn,),
        in_specs=[
            pl.BlockSpec((1, bn, 16), lambda i: (0, i, 0)),
            pl.BlockSpec((1, bn, 16), lambda i: (1, i, 0)),
            pl.BlockSpec((1, bn, 16), lambda i: (0, i, 0)),
            pl.BlockSpec((1, bn, 16), lambda i: (1, i, 0)),
            pl.BlockSpec((1, 32), lambda i: (0, 0)),
            pl.BlockSpec((32, 64), lambda i: (0, 0)),
            pl.BlockSpec((32, 64), lambda i: (0, 0)),
            pl.BlockSpec((64, 16), lambda i: (0, 0)),
            pl.BlockSpec((64, 16), lambda i: (0, 0)),
        ],
        out_specs=[
            pl.BlockSpec((bn, 64), lambda i: (i, 0)),
            pl.BlockSpec((bn, 64), lambda i: (i, 0)),
            pl.BlockSpec((bn, 16), lambda i: (i, 0)),
            pl.BlockSpec((8, 16), lambda i: (0, 0)),
        ],
        out_shape=[
            jax.ShapeDtypeStruct((n, 64), _f32),
            jax.ShapeDtypeStruct((n, 64), _f32),
            jax.ShapeDtypeStruct((n_pad, 16), _f32),
            jax.ShapeDtypeStruct((8, 16), _f32),
        ],
    )(slo, slo, shi, shi, b0r, w1lo, w1hi, amlo, amhi)


def _tc_final(p0, p1, w_out, b_out_r):
    out = w_out.shape[1]

    def body(p0_ref, p1_ref, w_ref, b_ref, y_ref):
        g = p0_ref[...] + p1_ref[...]
        y_ref[...] = jnp.dot(g, w_ref[...],
                             preferred_element_type=_f32) + b_ref[...]

    return pl.pallas_call(
        body,
        out_shape=jax.ShapeDtypeStruct((G, out), _f32),
    )(p0, p1, w_out, b_out_r)


def _head_tables(thw, a_src, a_dst):
    r = thw.shape[0]
    t3 = thw.reshape(r, H, C)
    as_t = jnp.einsum("nhc,hc->nh", t3, a_src)
    ad_t = jnp.einsum("nhc,hc->nh", t3, a_dst)
    a_tab = jnp.concatenate([as_t, ad_t, jnp.zeros((r, 8), _f32)], axis=1)
    return as_t, ad_t, a_tab


def _mvec(mx_as, mx_ad):
    m = jnp.maximum(mx_as + mx_ad, 0.0)
    return jnp.tile(m, 4)


def _amat_half(a_src_h, a_dst_h):
    hs = jnp.arange(H)
    src_m = jnp.zeros((H, L, H), _f32).at[hs, :, hs].set(a_src_h)
    dst_m = jnp.zeros((H, L, H), _f32).at[hs, :, hs].set(a_dst_h)
    return jnp.concatenate(
        [src_m.reshape(H * L, H), dst_m.reshape(H * L, H),
         jnp.zeros((H * L, 8), _f32)], axis=1)


def _half_cols(w):
    w3 = w.reshape(C, H, C)
    return (w3[:, :, :L].reshape(C, H * L), w3[:, :, L:].reshape(C, H * L))


def kernel(x, edge_index, batch, emb, W0, W1, a_src0, a_src1, a_dst0,
           a_dst1, b0, b1, W_out, b_out):
    n = x.shape[0]
    k = 128
    n_pad = ((n + k - 1) // k) * k

    idx = x[:, 0].astype(_i32)
    idx_pad = jnp.concatenate([idx, jnp.zeros((n_pad - n,), _i32)])
    src = edge_index[0]
    dst = edge_index[1]
    batch_pad = jnp.concatenate([batch, jnp.zeros((n_pad - n,), _i32)])

    thw0 = jnp.dot(emb, W0, preferred_element_type=_f32)
    tlo0, thi0 = _half_cols(W0)
    tlo0 = jnp.dot(emb, tlo0, preferred_element_type=_f32)
    thi0 = jnp.dot(emb, thi0, preferred_element_type=_f32)
    as0, ad0, ta0 = _head_tables(thw0, a_src0, a_dst0)
    mv0 = _mvec(jnp.max(as0, axis=0), jnp.max(ad0, axis=0))

    hw0lo, hw0hi, a0 = _pass0(idx_pad, tlo0, thi0, ta0, np_pad=n_pad, k=k)
    den1 = _pass1(src, dst, a0, mv0, n=n, n_pad=n_pad, k=k)
    d1 = _combine(den1, a0, k=k)
    s1lo = _pass2h(src, dst, a0, d1, hw0lo, mv0, n=n, n_pad=n_pad, k=k)
    s1hi = _pass2h(src, dst, a0, d1, hw0hi, mv0, n=n, n_pad=n_pad, k=k)

    w1lo, w1hi = _half_cols(W1)
    amlo = _amat_half(a_src1[:, :L], a_dst1[:, :L])
    amhi = _amat_half(a_src1[:, L:], a_dst1[:, L:])
    hw1lo, hw1hi, a1, mx = _tc_layer(
        s1lo, s1hi, b0.reshape(1, C), w1lo, w1hi, amlo, amhi,
        n=n, n_pad=n_pad, bn=1000)
    mv1 = _mvec(mx[0, 0:4], mx[0, 4:8])

    den2 = _pass1(src, dst, a1, mv1, n=n, n_pad=n_pad, k=k)
    d2 = _combine(den2, a1, k=k)
    s2lo = _pass2h(src, dst, a1, d2, hw1lo, mv1, n=n, n_pad=n_pad, k=k)
    s2hi = _pass2h(src, dst, a1, d2, hw1hi, mv1, n=n, n_pad=n_pad, k=k)

    pooled = _pool(s2lo, s2hi, batch_pad, b1, n=n, k=k)
    return _tc_final(pooled[0], pooled[1], W_out, b_out.reshape(1, -1))


# speedup vs baseline: 1.0056x; 1.0056x over previous
import functools

import jax
import jax.numpy as jnp
from jax import lax
from jax.experimental import pallas as pl
from jax.experimental.pallas import tpu as pltpu
from jax.experimental.pallas import tpu_sc as plsc

_f32 = jnp.float32
_i32 = jnp.int32

NC = 2
NS = 16
NW = NC * NS
L = 16

H = 4
C = 32
G = 512

_DNUMS = lax.GatherDimensionNumbers(
    offset_dims=(), collapsed_slice_dims=(0,), start_index_map=(0,))


def _lane_shuffle(v, idx):
    return lax.gather(v, idx[:, None], _DNUMS, slice_sizes=(1,),
                      mode=lax.GatherScatterMode.PROMISE_IN_BOUNDS)


def _shift4(v):
    io = lax.iota(_i32, L)
    return _lane_shuffle(v, jnp.minimum(io + 4, 8))


def _splat(v, lane):
    return _lane_shuffle(v, lax.iota(_i32, L) * 0 + lane)


def _mesh():
    return plsc.VectorSubcoreMesh(core_axis_name="c", subcore_axis_name="s")


_SC_PARAMS = pltpu.CompilerParams(use_tc_tiling_on_sc=False)


def _zero_vmem(ref, rows, width):
    z = jnp.zeros((L,), _f32)

    @plsc.parallel_loop(0, rows, unroll=4)
    def _(r):
        for kk in range(width // L):
            ref[r, pl.ds(kk * L, L)] = z


@functools.partial(jax.jit, static_argnames=("np_pad", "k"))
def _pass0(idx, tlo, thi, ta, *, np_pad, k):
    nchunks = np_pad // k

    @functools.partial(
        pl.kernel,
        out_type=(jax.ShapeDtypeStruct((np_pad, 64), _f32),
                  jax.ShapeDtypeStruct((np_pad, 64), _f32),
                  jax.ShapeDtypeStruct((np_pad, 16), _f32)),
        mesh=_mesh(),
        compiler_params=_SC_PARAMS,
        scratch_types=[
            pltpu.VMEM((2, k), _i32),
            pltpu.VMEM((2, k, 64), _f32),
            pltpu.VMEM((2, k, 64), _f32),
            pltpu.VMEM((2, k, 16), _f32),
            pltpu.SemaphoreType.DMA,
            pltpu.SemaphoreType.DMA,
        ],
    )
    def kfn(idx_hbm, tlo_hbm, thi_hbm, ta_hbm, lo_out, hi_out, a_out,
            idx_v, lo_v, hi_v, a_v, sg0, sg1):
        w = lax.axis_index("s") * NC + lax.axis_index("c")
        trips = (nchunks - w + NW - 1) // NW
        sgs = (sg0, sg1)

        def prep(ci, b):
            base = (w + ci * NW) * k
            pltpu.sync_copy(idx_hbm.at[pl.ds(base, k)], idx_v.at[b])
            pltpu.async_copy(tlo_hbm.at[idx_v.at[b]], lo_v.at[b], sgs[b])
            pltpu.async_copy(thi_hbm.at[idx_v.at[b]], hi_v.at[b], sgs[b])
            pltpu.async_copy(ta_hbm.at[idx_v.at[b]], a_v.at[b], sgs[b])

        def wait_gathers(b):
            pltpu.make_async_copy(
                tlo_hbm.at[idx_v.at[b]], lo_v.at[b], sgs[b]).wait()
            pltpu.make_async_copy(
                thi_hbm.at[idx_v.at[b]], hi_v.at[b], sgs[b]).wait()
            pltpu.make_async_copy(
                ta_hbm.at[idx_v.at[b]], a_v.at[b], sgs[b]).wait()

        @pl.when(trips > 0)
        def _():
            prep(0, 0)

        def body(p, _):
            for b in range(2):
                ci = 2 * p + b

                @pl.when(ci < trips)
                def _():
                    @pl.when(ci + 1 < trips)
                    def _():
                        prep(ci + 1, 1 - b)

                    wait_gathers(b)
                    base = (w + ci * NW) * k
                    pltpu.sync_copy(lo_v.at[b], lo_out.at[pl.ds(base, k)])
                    pltpu.sync_copy(hi_v.at[b], hi_out.at[pl.ds(base, k)])
                    pltpu.sync_copy(a_v.at[b], a_out.at[pl.ds(base, k)])
            return 0

        lax.fori_loop(0, (trips + 1) // 2, body, 0)

    return kfn(idx, tlo, thi, ta)


@functools.partial(jax.jit, static_argnames=("n", "n_pad", "k"))
def _pass1(src, dst, a_tab, mvec, *, n, n_pad, k):
    e_tot = src.shape[0]
    nchunks = e_tot // k
    rpt = n // NS
    zr = 625
    assert rpt % zr == 0

    @functools.partial(
        pl.kernel,
        out_type=jax.ShapeDtypeStruct((NC, n_pad, 16), _f32),
        mesh=_mesh(),
        compiler_params=_SC_PARAMS,
        scratch_types=[
            pltpu.VMEM((2, k), _i32),
            pltpu.VMEM((2, k), _i32),
            pltpu.VMEM((2, k), _i32),
            pltpu.VMEM((2, k, 16), _f32),
            pltpu.VMEM((2, k, 16), _f32),
            pltpu.VMEM((2, k, 16), _f32),
            pltpu.VMEM((zr, 16), _f32),
            pltpu.VMEM((L,), _f32),
            pltpu.VMEM_SHARED((n, 16), _f32),
            pltpu.SemaphoreType.DMA,
            pltpu.SemaphoreType.DMA,
            pltpu.SemaphoreType.DMA,
            pltpu.SemaphoreType.DMA,
        ],
    )
    def kfn(src_hbm, dst_hbm, a_hbm, m_hbm, den_out,
            src_v, dst_v, dsc_v, va_v, vb_v, t_v, z_v, m_v, den_sh,
            sg0, sg1, ss0, ss1):
        c = lax.axis_index("c")
        s = lax.axis_index("s")
        w = s * NC + c
        sgs = (sg0, sg1)
        sss = (ss0, ss1)
        _zero_vmem(z_v, zr, 16)
        for j in range(rpt // zr):
            pltpu.sync_copy(z_v, den_sh.at[pl.ds(s * rpt + j * zr, zr)])
        pltpu.sync_copy(m_hbm, m_v)
        mv = m_v[...]
        plsc.subcore_barrier()
        trips = (nchunks - w + NW - 1) // NW

        def load_idx(ci, b):
            base = (w + ci * NW) * k
            pltpu.sync_copy(src_hbm.at[pl.ds(base, k)], src_v.at[b])
            pltpu.sync_copy(dst_hbm.at[pl.ds(base, k)], dst_v.at[b])

        def issue_gathers(b):
            pltpu.async_copy(a_hbm.at[src_v.at[b]], va_v.at[b], sgs[b])
            pltpu.async_copy(a_hbm.at[dst_v.at[b]], vb_v.at[b], sgs[b])

        def wait_gathers(b):
            pltpu.make_async_copy(
                a_hbm.at[src_v.at[b]], va_v.at[b], sgs[b]).wait()
            pltpu.make_async_copy(
                a_hbm.at[dst_v.at[b]], vb_v.at[b], sgs[b]).wait()

        def issue_scatter(b):
            pltpu.async_copy(t_v.at[b], den_sh.at[dsc_v.at[b]], sss[b],
                             add=True)

        def wait_scatter(b):
            pltpu.make_async_copy(
                t_v.at[b], den_sh.at[dsc_v.at[b]], sss[b]).wait()

        def compute(b):
            @plsc.parallel_loop(0, k // L, unroll=2)
            def _(r):
                dsc_v[b, pl.ds(r * L, L)] = dst_v[b, pl.ds(r * L, L)]

            @plsc.parallel_loop(0, k, unroll=8)
            def _(e):
                va = va_v[b, e, :]
                vb = vb_v[b, e, :]
                ev = va + _shift4(vb)
                lr = jnp.maximum(ev, 0.2 * ev)
                t_v[b, e, :] = jnp.exp(lr - mv)

        @pl.when(trips > 0)
        def _():
            load_idx(0, 0)
            issue_gathers(0)

        def body(p, _):
            for b in range(2):
                ci = 2 * p + b

                @pl.when(ci < trips)
                def _():
                    @pl.when(ci + 1 < trips)
                    def _():
                        load_idx(ci + 1, 1 - b)
                        issue_gathers(1 - b)

                    wait_gathers(b)

                    @pl.when(ci >= 2)
                    def _():
                        wait_scatter(b)

                    compute(b)
                    issue_scatter(b)
            return 0

        lax.fori_loop(0, (trips + 1) // 2, body, 0)

        def wait_scatter_dyn(j):
            @pl.when(j >= 0)
            def _():
                @pl.when(lax.rem(j, 2) == 0)
                def _():
                    wait_scatter(0)

                @pl.when(lax.rem(j, 2) == 1)
                def _():
                    wait_scatter(1)

        wait_scatter_dyn(trips - 1)
        wait_scatter_dyn(trips - 2)
        plsc.subcore_barrier()
        for j in range(rpt // zr):
            off = s * rpt + j * zr
            pltpu.sync_copy(den_sh.at[pl.ds(off, zr)],
                            den_out.at[c, pl.ds(off, zr)])

    return kfn(src, dst, a_tab, mvec)


@functools.partial(jax.jit, static_argnames=("k",))
def _combine(den, a_tab, *, k):
    n_pad = a_tab.shape[0]
    nchunks = n_pad // k

    @functools.partial(
        pl.kernel,
        out_type=jax.ShapeDtypeStruct((n_pad, 16), _f32),
        mesh=_mesh(),
        compiler_params=_SC_PARAMS,
        scratch_types=[
            pltpu.VMEM((k, 16), _f32),
            pltpu.VMEM((k, 16), _f32),
            pltpu.VMEM((k, 16), _f32),
            pltpu.VMEM((k, 16), _f32),
        ],
    )
    def kfn(den_hbm, a_hbm, d_out, d0_v, d1_v, a_v, o_v):
        w = lax.axis_index("s") * NC + lax.axis_index("c")
        trips = (nchunks - w + NW - 1) // NW
        io = lax.iota(_i32, L)
        iof = io.astype(_f32)
        m47 = jnp.maximum(
            0.0, jnp.minimum(1.0, jnp.minimum(iof - 3.0, 8.0 - iof)))
        sh_r = jnp.maximum(io - 4, 0)

        def body(i, _):
            base = (w + i * NW) * k
            pltpu.sync_copy(den_hbm.at[0, pl.ds(base, k)], d0_v)
            pltpu.sync_copy(den_hbm.at[1, pl.ds(base, k)], d1_v)
            pltpu.sync_copy(a_hbm.at[pl.ds(base, k)], a_v)

            @plsc.parallel_loop(0, k, unroll=4)
            def _(r):
                sm = d0_v[r, :] + d1_v[r, :] + 1e-16
                rr = 1.0 / sm
                rsh = _lane_shuffle(rr, sh_r)
                o_v[r, :] = _shift4(a_v[r, :]) + rsh * m47

            pltpu.sync_copy(o_v, d_out.at[pl.ds(base, k)])
            return 0

        lax.fori_loop(0, trips, body, 0)

    return kfn(den, a_tab)


@functools.partial(jax.jit, static_argnames=("n", "n_pad", "k"))
def _pass2h(src, dst, a_tab, d_tab, hwh, mvec, *, n, n_pad, k):
    e_tot = src.shape[0]
    nchunks = e_tot // k
    rpt = n // NS
    zr = 625
    assert rpt % zr == 0

    @functools.partial(
        pl.kernel,
        out_type=jax.ShapeDtypeStruct((NC, n_pad, 16), _f32),
        mesh=_mesh(),
        compiler_params=_SC_PARAMS,
        scratch_types=[
            pltpu.VMEM((2, k), _i32),
            pltpu.VMEM((2, k), _i32),
            pltpu.VMEM((2, k), _i32),
            pltpu.VMEM((2, k, 16), _f32),
            pltpu.VMEM((2, k, 16), _f32),
            pltpu.VMEM((2, k, 64), _f32),
            pltpu.VMEM((2, k, 16), _f32),
            pltpu.VMEM((zr, 16), _f32),
            pltpu.VMEM((L,), _f32),
            pltpu.VMEM_SHARED((n, 16), _f32),
            pltpu.SemaphoreType.DMA,
            pltpu.SemaphoreType.DMA,
            pltpu.SemaphoreType.DMA,
            pltpu.SemaphoreType.DMA,
        ],
    )
    def kfn(src_hbm, dst_hbm, a_hbm, d_hbm, hw_hbm, m_hbm, out_hbm,
            src_v, dst_v, dsc_v, va_v, vd_v, hw_v, c_v, z_v, m_v, acc_sh,
            sg0, sg1, ss0, ss1):
        c = lax.axis_index("c")
        s = lax.axis_index("s")
        w = s * NC + c
        sgs = (sg0, sg1)
        sss = (ss0, ss1)
        _zero_vmem(z_v, zr, 16)
        for j in range(rpt // zr):
            pltpu.sync_copy(z_v, acc_sh.at[pl.ds(s * rpt + j * zr, zr)])
        pltpu.sync_copy(m_hbm, m_v)
        mv = m_v[...]
        plsc.subcore_barrier()
        trips = (nchunks - w + NW - 1) // NW

        def load_idx(ci, b):
            base = (w + ci * NW) * k
            pltpu.sync_copy(src_hbm.at[pl.ds(base, k)], src_v.at[b])
            pltpu.sync_copy(dst_hbm.at[pl.ds(base, k)], dst_v.at[b])

        def issue_gathers(b):
            pltpu.async_copy(a_hbm.at[src_v.at[b]], va_v.at[b], sgs[b])
            pltpu.async_copy(d_hbm.at[dst_v.at[b]], vd_v.at[b], sgs[b])
            pltpu.async_copy(hw_hbm.at[src_v.at[b]], hw_v.at[b], sgs[b])

        def wait_gathers(b):
            pltpu.make_async_copy(
                a_hbm.at[src_v.at[b]], va_v.at[b], sgs[b]).wait()
            pltpu.make_async_copy(
                d_hbm.at[dst_v.at[b]], vd_v.at[b], sgs[b]).wait()
            pltpu.make_async_copy(
                hw_hbm.at[src_v.at[b]], hw_v.at[b], sgs[b]).wait()

        def issue_scatter(b):
            pltpu.async_copy(c_v.at[b], acc_sh.at[dsc_v.at[b]], sss[b],
                             add=True)

        def wait_scatter(b):
            pltpu.make_async_copy(
                c_v.at[b], acc_sh.at[dsc_v.at[b]], sss[b]).wait()

        def compute(b):
            @plsc.parallel_loop(0, k // L, unroll=2)
            def _(r):
                dsc_v[b, pl.ds(r * L, L)] = dst_v[b, pl.ds(r * L, L)]

            @plsc.parallel_loop(0, k, unroll=4)
            def _(e):
                va = va_v[b, e, :]
                vd = vd_v[b, e, :]
                ev = va + vd
                lr = jnp.maximum(ev, 0.2 * ev)
                t = jnp.exp(lr - mv)
                alpha = t * _shift4(vd)
                acc = _splat(alpha, 0) * hw_v[b, e, pl.ds(0, L)]
                for h in range(1, H):
                    acc = acc + _splat(alpha, h) * hw_v[b, e, pl.ds(h * L, L)]
                c_v[b, e, :] = acc

        @pl.when(trips > 0)
        def _():
            load_idx(0, 0)
            issue_gathers(0)

        def body(p, _):
            for b in range(2):
                ci = 2 * p + b

                @pl.when(ci < trips)
                def _():
                    @pl.when(ci + 1 < trips)
                    def _():
                        load_idx(ci + 1, 1 - b)
                        issue_gathers(1 - b)

                    wait_gathers(b)

                    @pl.when(ci >= 2)
                    def _():
                        wait_scatter(b)

                    compute(b)
                    issue_scatter(b)
            return 0

        lax.fori_loop(0, (trips + 1) // 2, body, 0)

        def wait_scatter_dyn(j):
            @pl.when(j >= 0)
            def _():
                @pl.when(lax.rem(j, 2) == 0)
                def _():
                    wait_scatter(0)

                @pl.when(lax.rem(j, 2) == 1)
                def _():
                    wait_scatter(1)

        wait_scatter_dyn(trips - 1)
        wait_scatter_dyn(trips - 2)
        plsc.subcore_barrier()
        for j in range(rpt // zr):
            off = s * rpt + j * zr
            pltpu.sync_copy(acc_sh.at[pl.ds(off, zr)],
                            out_hbm.at[c, pl.ds(off, zr)])

    return kfn(src, dst, a_tab, d_tab, hwh, mvec)


@functools.partial(jax.jit, static_argnames=("n", "k"))
def _pool(s2lo, s2hi, batch, b1, *, n, k):
    n_pad = batch.shape[0]
    nchunks = n_pad // k
    rpt = G // NS

    @functools.partial(
        pl.kernel,
        out_type=jax.ShapeDtypeStruct((NC, G, 32), _f32),
        mesh=_mesh(),
        compiler_params=_SC_PARAMS,
        scratch_types=[
            pltpu.VMEM((k,), _i32),
            pltpu.VMEM((k, 16), _f32),
            pltpu.VMEM((k, 16), _f32),
            pltpu.VMEM((k, 16), _f32),
            pltpu.VMEM((k, 16), _f32),
            pltpu.VMEM((k, 32), _f32),
            pltpu.VMEM((rpt, 32), _f32),
            pltpu.VMEM((32,), _f32),
            pltpu.VMEM_SHARED((G, 32), _f32),
        ],
    )
    def kfn(lo_hbm, hi_hbm, bat_hbm, b1_hbm, out_hbm,
            bat_v, l0_v, l1_v, h0_v, h1_v, h_v, z_v, b_v, acc_sh):
        c = lax.axis_index("c")
        s = lax.axis_index("s")
        w = s * NC + c
        _zero_vmem(z_v, rpt, 32)
        pltpu.sync_copy(z_v, acc_sh.at[pl.ds(s * rpt, rpt)])
        pltpu.sync_copy(b1_hbm, b_v)
        blo = b_v[pl.ds(0, L)]
        bhi = b_v[pl.ds(L, L)]
        plsc.subcore_barrier()
        trips = (nchunks - w + NW - 1) // NW
        inv_h = _f32(1.0 / H)

        def body(i, _):
            base = (w + i * NW) * k
            pltpu.sync_copy(bat_hbm.at[pl.ds(base, k)], bat_v)
            pltpu.sync_copy(lo_hbm.at[0, pl.ds(base, k)], l0_v)
            pltpu.sync_copy(lo_hbm.at[1, pl.ds(base, k)], l1_v)
            pltpu.sync_copy(hi_hbm.at[0, pl.ds(base, k)], h0_v)
            pltpu.sync_copy(hi_hbm.at[1, pl.ds(base, k)], h1_v)

            @plsc.parallel_loop(0, k, unroll=4)
            def _(r):
                valid = jnp.where(base + r < n, _f32(1.0), _f32(0.0))
                lo = (l0_v[r, :] + l1_v[r, :]) * inv_h
                hi = (h0_v[r, :] + h1_v[r, :]) * inv_h
                lo = jnp.maximum(lo + blo, 0.0)
                hi = jnp.maximum(hi + bhi, 0.0)
                h_v[r, pl.ds(0, L)] = lo * valid
                h_v[r, pl.ds(L, L)] = hi * valid
            pltpu.sync_copy(h_v, acc_sh.at[bat_v], add=True)
            return 0

        lax.fori_loop(0, trips, body, 0)
        plsc.subcore_barrier()
        pltpu.sync_copy(acc_sh.at[pl.ds(s * rpt, rpt)],
                        out_hbm.at[c, pl.ds(s * rpt, rpt)])

    return kfn(s2lo, s2hi, batch, b1)


def _tc_layer(slo, shi, b0r, w1lo, w1hi, amlo, amhi, *, n, n_pad, bn=1000):
    def body(sl0_ref, sl1_ref, sh0_ref, sh1_ref, b_ref, wl_ref, wh_ref,
             al_ref, ah_ref, lo_ref, hi_ref, a_ref, mx_ref):
        hmean = jnp.concatenate(
            [sl0_ref[0] + sl1_ref[0], sh0_ref[0] + sh1_ref[0]],
            axis=1) * _f32(1.0 / H)
        h = jnp.maximum(hmean + b_ref[...], 0.0)
        hwlo = jnp.dot(h, wl_ref[...], preferred_element_type=_f32)
        hwhi = jnp.dot(h, wh_ref[...], preferred_element_type=_f32)
        a1 = (jnp.dot(hwlo, al_ref[...], preferred_element_type=_f32)
              + jnp.dot(hwhi, ah_ref[...], preferred_element_type=_f32))
        lo_ref[...] = hwlo
        hi_ref[...] = hwhi
        a_ref[...] = a1
        bm = jnp.broadcast_to(jnp.max(a1, axis=0, keepdims=True), (8, 16))
        i = pl.program_id(0)

        @pl.when(i == 0)
        def _():
            mx_ref[...] = bm

        @pl.when(i > 0)
        def _():
            mx_ref[...] = jnp.maximum(mx_ref[...], bm)

    return pl.pallas_call(
        body,
        grid=(n // bn,),
        in_specs=[
            pl.BlockSpec((1, bn, 16), lambda i: (0, i, 0)),
            pl.BlockSpec((1, bn, 16), lambda i: (1, i, 0)),
            pl.BlockSpec((1, bn, 16), lambda i: (0, i, 0)),
            pl.BlockSpec((1, bn, 16), lambda i: (1, i, 0)),
            pl.BlockSpec((1, 32), lambda i: (0, 0)),
            pl.BlockSpec((32, 64), lambda i: (0, 0)),
            pl.BlockSpec((32, 64), lambda i: (0, 0)),
            pl.BlockSpec((64, 16), lambda i: (0, 0)),
            pl.BlockSpec((64, 16), lambda i: (0, 0)),
        ],
        out_specs=[
            pl.BlockSpec((bn, 64), lambda i: (i, 0)),
            pl.BlockSpec((bn, 64), lambda i: (i, 0)),
            pl.BlockSpec((bn, 16), lambda i: (i, 0)),
            pl.BlockSpec((8, 16), lambda i: (0, 0)),
        ],
        out_shape=[
            jax.ShapeDtypeStruct((n, 64), _f32),
            jax.ShapeDtypeStruct((n, 64), _f32),
            jax.ShapeDtypeStruct((n_pad, 16), _f32),
            jax.ShapeDtypeStruct((8, 16), _f32),
        ],
    )(slo, slo, shi, shi, b0r, w1lo, w1hi, amlo, amhi)


def _tc_final(p0, p1, w_out, b_out_r):
    out = w_out.shape[1]

    def body(p0_ref, p1_ref, w_ref, b_ref, y_ref):
        g = p0_ref[...] + p1_ref[...]
        y_ref[...] = jnp.dot(g, w_ref[...],
                             preferred_element_type=_f32) + b_ref[...]

    return pl.pallas_call(
        body,
        out_shape=jax.ShapeDtypeStruct((G, out), _f32),
    )(p0, p1, w_out, b_out_r)


def _head_tables(thw, a_src, a_dst):
    r = thw.shape[0]
    t3 = thw.reshape(r, H, C)
    as_t = jnp.einsum("nhc,hc->nh", t3, a_src)
    ad_t = jnp.einsum("nhc,hc->nh", t3, a_dst)
    a_tab = jnp.concatenate([as_t, ad_t, jnp.zeros((r, 8), _f32)], axis=1)
    return as_t, ad_t, a_tab


def _mvec(mx_as, mx_ad):
    m = jnp.maximum(mx_as + mx_ad, 0.0)
    return jnp.tile(m, 4)


def _amat_half(a_src_h, a_dst_h):
    hs = jnp.arange(H)
    src_m = jnp.zeros((H, L, H), _f32).at[hs, :, hs].set(a_src_h)
    dst_m = jnp.zeros((H, L, H), _f32).at[hs, :, hs].set(a_dst_h)
    return jnp.concatenate(
        [src_m.reshape(H * L, H), dst_m.reshape(H * L, H),
         jnp.zeros((H * L, 8), _f32)], axis=1)


def _half_cols(w):
    w3 = w.reshape(C, H, C)
    return (w3[:, :, :L].reshape(C, H * L), w3[:, :, L:].reshape(C, H * L))


def kernel(x, edge_index, batch, emb, W0, W1, a_src0, a_src1, a_dst0,
           a_dst1, b0, b1, W_out, b_out):
    n = x.shape[0]
    k = 128
    n_pad = ((n + k - 1) // k) * k

    idx = x[:, 0].astype(_i32)
    idx_pad = jnp.concatenate([idx, jnp.zeros((n_pad - n,), _i32)])
    src = edge_index[0]
    dst = edge_index[1]
    batch_pad = jnp.concatenate([batch, jnp.zeros((n_pad - n,), _i32)])

    thw0 = jnp.dot(emb, W0, preferred_element_type=_f32)
    tlo0, thi0 = _half_cols(W0)
    tlo0 = jnp.dot(emb, tlo0, preferred_element_type=_f32)
    thi0 = jnp.dot(emb, thi0, preferred_element_type=_f32)
    as0, ad0, ta0 = _head_tables(thw0, a_src0, a_dst0)
    mv0 = _mvec(jnp.max(as0, axis=0), jnp.max(ad0, axis=0))

    hw0lo, hw0hi, a0 = _pass0(idx_pad, tlo0, thi0, ta0, np_pad=n_pad, k=k)
    den1 = _pass1(src, dst, a0, mv0, n=n, n_pad=n_pad, k=k)
    d1 = _combine(den1, a0, k=k)
    s1lo = _pass2h(src, dst, a0, d1, hw0lo, mv0, n=n, n_pad=n_pad, k=k)
    s1hi = _pass2h(src, dst, a0, d1, hw0hi, mv0, n=n, n_pad=n_pad, k=k)

    w1lo, w1hi = _half_cols(W1)
    amlo = _amat_half(a_src1[:, :L], a_dst1[:, :L])
    amhi = _amat_half(a_src1[:, L:], a_dst1[:, L:])
    hw1lo, hw1hi, a1, mx = _tc_layer(
        s1lo, s1hi, b0.reshape(1, C), w1lo, w1hi, amlo, amhi,
        n=n, n_pad=n_pad, bn=1000)
    mv1 = _mvec(mx[0, 0:4], mx[0, 4:8])

    den2 = _pass1(src, dst, a1, mv1, n=n, n_pad=n_pad, k=k)
    d2 = _combine(den2, a1, k=k)
    s2lo = _pass2h(src, dst, a1, d2, hw1lo, mv1, n=n, n_pad=n_pad, k=k)
    s2hi = _pass2h(src, dst, a1, d2, hw1hi, mv1, n=n, n_pad=n_pad, k=k)

    pooled = _pool(s2lo, s2hi, batch_pad, b1, n=n, k=k)
    return _tc_final(pooled[0], pooled[1], W_out, b_out.reshape(1, -1))
